# Initial kernel scaffold; baseline (speedup 1.0000x reference)
import jax
import jax.numpy as jnp
from jax.experimental import pallas as pl


def kernel(local_energies, Z, batch, shift, scale):
    raise NotImplementedError("write your pallas kernel here")


# speedup vs baseline: 192.2365x; 192.2365x over previous
import functools

import jax
import jax.numpy as jnp
from jax import lax
from jax.experimental import pallas as pl
from jax.experimental.pallas import tpu as pltpu
from jax.experimental.pallas import tpu_sc as plsc

N_STRUCTURES = 1024
TAB_PAD = 128
LANES = 16


def _make_sc_partials(n_atoms):
    info = plsc.get_sparse_core_info()
    nc, ns = info.num_cores, info.num_subcores
    nw = nc * ns
    per_w = n_atoms // nw
    assert per_w * nw == n_atoms and per_w % LANES == 0

    chunk = per_w
    for cand in (12800, 10000, 6400, 3200, 1600, 800, 400, 80, 16):
        if per_w % cand == 0:
            chunk = cand
            break
    n_chunks = per_w // chunk
    vec_per_chunk = chunk // LANES

    mesh = plsc.VectorSubcoreMesh(core_axis_name="c", subcore_axis_name="s")

    @functools.partial(
        pl.kernel,
        mesh=mesh,
        out_type=jax.ShapeDtypeStruct((nw, N_STRUCTURES), jnp.float32),
        compiler_params=pltpu.CompilerParams(needs_layout_passes=False),
        scratch_types=[
            pltpu.VMEM((chunk,), jnp.float32),
            pltpu.VMEM((chunk,), jnp.int32),
            pltpu.VMEM((chunk,), jnp.int32),
            pltpu.VMEM((TAB_PAD,), jnp.float32),
            pltpu.VMEM((TAB_PAD,), jnp.float32),
            pltpu.VMEM((N_STRUCTURES,), jnp.float32),
        ],
    )
    def sc_kernel(e_hbm, z_hbm, b_hbm, scale_hbm, shift_hbm, out_hbm,
                  e_v, z_v, b_v, scale_v, shift_v, acc_v):
        wid = lax.axis_index("s") * nc + lax.axis_index("c")
        base = wid * per_w

        pltpu.sync_copy(scale_hbm, scale_v)
        pltpu.sync_copy(shift_hbm, shift_v)

        def zero_body(i, _):
            acc_v[pl.ds(i * LANES, LANES)] = jnp.zeros((LANES,), jnp.float32)
            return 0

        lax.fori_loop(0, N_STRUCTURES // LANES, zero_body, 0)

        def do_chunk(ci, _):
            start = base + ci * chunk
            pltpu.sync_copy(e_hbm.at[pl.ds(start, chunk)], e_v)
            pltpu.sync_copy(z_hbm.at[pl.ds(start, chunk)], z_v)
            pltpu.sync_copy(b_hbm.at[pl.ds(start, chunk)], b_v)

            def vec_body(i, _):
                off = i * LANES
                e = e_v[pl.ds(off, LANES)]
                z = z_v[pl.ds(off, LANES)]
                b = b_v[pl.ds(off, LANES)]
                sc = plsc.load_gather(scale_v, [z])
                sh = plsc.load_gather(shift_v, [z])
                plsc.addupdate_scatter(acc_v, [b], e * sc + sh)
                return 0

            lax.fori_loop(0, vec_per_chunk, vec_body, 0)
            return 0

        lax.fori_loop(0, n_chunks, do_chunk, 0)
        pltpu.sync_copy(acc_v, out_hbm.at[wid])

    return sc_kernel, nw


def _sum_partials_body(p_ref, o_ref):
    o_ref[:] = jnp.sum(p_ref[:], axis=0)


def kernel(local_energies, Z, batch, shift, scale):
    n_atoms = local_energies.shape[0]
    scale_p = jnp.zeros((TAB_PAD,), jnp.float32).at[: scale.shape[0]].set(scale)
    shift_p = jnp.zeros((TAB_PAD,), jnp.float32).at[: shift.shape[0]].set(shift)

    sc_kernel, nw = _make_sc_partials(n_atoms)
    partials = sc_kernel(local_energies, Z, batch, scale_p, shift_p)

    total = pl.pallas_call(
        _sum_partials_body,
        out_shape=jax.ShapeDtypeStruct((N_STRUCTURES,), jnp.float32),
    )(partials)
    return total


# speedup vs baseline: 224.9389x; 1.1701x over previous
import functools

import jax
import jax.numpy as jnp
from jax import lax
from jax.experimental import pallas as pl
from jax.experimental.pallas import tpu as pltpu
from jax.experimental.pallas import tpu_sc as plsc

N_STRUCTURES = 1024
TAB_PAD = 128
LANES = 16


def _make_sc_partials(n_atoms):
    info = plsc.get_sparse_core_info()
    nc, ns = info.num_cores, info.num_subcores
    nw = nc * ns
    per_w = n_atoms // nw
    assert per_w * nw == n_atoms and per_w % LANES == 0

    chunk = per_w
    for cand in (10000, 6400, 3200, 1600, 800, 400, 80, 16):
        if per_w % cand == 0:
            chunk = cand
            break
    n_chunks = per_w // chunk
    vec_per_chunk = chunk // LANES
    unroll = 1
    for u in (5, 4, 2):
        if vec_per_chunk % u == 0:
            unroll = u
            break

    mesh = plsc.VectorSubcoreMesh(core_axis_name="c", subcore_axis_name="s")

    @functools.partial(
        pl.kernel,
        mesh=mesh,
        out_type=jax.ShapeDtypeStruct((nw, N_STRUCTURES), jnp.float32),
        compiler_params=pltpu.CompilerParams(needs_layout_passes=False),
        scratch_types=[
            pltpu.VMEM((chunk,), jnp.float32),
            pltpu.VMEM((chunk,), jnp.int32),
            pltpu.VMEM((chunk,), jnp.int32),
            pltpu.VMEM((chunk,), jnp.float32),
            pltpu.VMEM((chunk,), jnp.int32),
            pltpu.VMEM((chunk,), jnp.int32),
            pltpu.VMEM((TAB_PAD,), jnp.float32),
            pltpu.VMEM((TAB_PAD,), jnp.float32),
            pltpu.VMEM((N_STRUCTURES,), jnp.float32),
            pltpu.SemaphoreType.DMA,
            pltpu.SemaphoreType.DMA,
        ],
    )
    def sc_kernel(e_hbm, z_hbm, b_hbm, scale_hbm, shift_hbm, out_hbm,
                  e0, z0, b0, e1, z1, b1, scale_v, shift_v, acc_v,
                  sem0, sem1):
        wid = lax.axis_index("s") * nc + lax.axis_index("c")
        base = wid * per_w
        bufs = ((e0, z0, b0), (e1, z1, b1))
        sems = (sem0, sem1)

        def start_chunk(ci, p):
            st = base + ci * chunk
            return (
                pltpu.async_copy(e_hbm.at[pl.ds(st, chunk)], bufs[p][0], sems[p]),
                pltpu.async_copy(z_hbm.at[pl.ds(st, chunk)], bufs[p][1], sems[p]),
                pltpu.async_copy(b_hbm.at[pl.ds(st, chunk)], bufs[p][2], sems[p]),
            )

        in_flight = start_chunk(0, 0)

        pltpu.sync_copy(scale_hbm, scale_v)
        pltpu.sync_copy(shift_hbm, shift_v)

        def zero_body(i, _):
            acc_v[pl.ds(i * LANES, LANES)] = jnp.zeros((LANES,), jnp.float32)
            return 0

        lax.fori_loop(0, N_STRUCTURES // LANES, zero_body, 0)

        for ci in range(n_chunks):
            p = ci % 2
            for d in in_flight:
                d.wait()
            if ci + 1 < n_chunks:
                in_flight = start_chunk(ci + 1, (ci + 1) % 2)
            e_v, z_v, b_v = bufs[p]

            def vec_body(i, _, e_v=e_v, z_v=z_v, b_v=b_v):
                for u in range(unroll):
                    off = (i * unroll + u) * LANES
                    e = e_v[pl.ds(off, LANES)]
                    z = z_v[pl.ds(off, LANES)]
                    b = b_v[pl.ds(off, LANES)]
                    sc = plsc.load_gather(scale_v, [z])
                    sh = plsc.load_gather(shift_v, [z])
                    plsc.addupdate_scatter(acc_v, [b], e * sc + sh)
                return 0

            lax.fori_loop(0, vec_per_chunk // unroll, vec_body, 0)

        pltpu.sync_copy(acc_v, out_hbm.at[wid])

    return sc_kernel, nw


def _sum_partials_body(p_ref, o_ref):
    o_ref[:] = jnp.sum(p_ref[:], axis=0)


def kernel(local_energies, Z, batch, shift, scale):
    n_atoms = local_energies.shape[0]
    scale_p = jnp.zeros((TAB_PAD,), jnp.float32).at[: scale.shape[0]].set(scale)
    shift_p = jnp.zeros((TAB_PAD,), jnp.float32).at[: shift.shape[0]].set(shift)

    sc_kernel, nw = _make_sc_partials(n_atoms)
    partials = sc_kernel(local_energies, Z, batch, scale_p, shift_p)

    total = pl.pallas_call(
        _sum_partials_body,
        out_shape=jax.ShapeDtypeStruct((N_STRUCTURES,), jnp.float32),
    )(partials)
    return total


# speedup vs baseline: 244.1636x; 1.0855x over previous
import functools

import jax
import jax.numpy as jnp
from jax import lax
from jax.experimental import pallas as pl
from jax.experimental.pallas import tpu as pltpu
from jax.experimental.pallas import tpu_sc as plsc

N_STRUCTURES = 1024
TAB_PAD = 128
LANES = 16
ROW = 128


def _make_sc_partials(n_atoms):
    info = plsc.get_sparse_core_info()
    nc, ns = info.num_cores, info.num_subcores
    nw = nc * ns
    assert n_atoms % ROW == 0
    n_rows = n_atoms // ROW
    rows_w = n_rows // nw
    tail_rows = n_rows - rows_w * nw
    assert tail_rows <= nw

    chunk_rows = rows_w
    for cand in (100, 96, 90, 78, 75, 65, 60, 50, 39, 30, 26, 15, 13, 10, 6, 5, 3, 2, 1):
        if rows_w % cand == 0:
            chunk_rows = cand
            break
    n_chunks = rows_w // chunk_rows
    vec_per_chunk = chunk_rows * (ROW // LANES)

    mesh = plsc.VectorSubcoreMesh(core_axis_name="c", subcore_axis_name="s")

    @functools.partial(
        pl.kernel,
        mesh=mesh,
        out_type=jax.ShapeDtypeStruct((nw, N_STRUCTURES), jnp.float32),
        compiler_params=pltpu.CompilerParams(
            needs_layout_passes=False, use_tc_tiling_on_sc=False
        ),
        scratch_types=[
            pltpu.VMEM((chunk_rows, ROW), jnp.float32),
            pltpu.VMEM((chunk_rows, ROW), jnp.int32),
            pltpu.VMEM((chunk_rows, ROW), jnp.int32),
            pltpu.VMEM((chunk_rows, ROW), jnp.float32),
            pltpu.VMEM((chunk_rows, ROW), jnp.int32),
            pltpu.VMEM((chunk_rows, ROW), jnp.int32),
            pltpu.VMEM_SHARED((ns, 2, chunk_rows, ROW), jnp.float32),
            pltpu.VMEM_SHARED((ns, 2, chunk_rows, ROW), jnp.int32),
            pltpu.VMEM_SHARED((ns, 2, chunk_rows, ROW), jnp.int32),
            pltpu.VMEM((1, ROW), jnp.float32),
            pltpu.VMEM((1, ROW), jnp.int32),
            pltpu.VMEM((1, ROW), jnp.int32),
            pltpu.VMEM((TAB_PAD,), jnp.float32),
            pltpu.VMEM((TAB_PAD,), jnp.float32),
            pltpu.VMEM((N_STRUCTURES,), jnp.float32),
            pltpu.SemaphoreType.DMA,
            pltpu.SemaphoreType.DMA,
            pltpu.SemaphoreType.DMA,
        ],
    )
    def sc_kernel(e_hbm, z_hbm, b_hbm, scale_hbm, shift_hbm, out_hbm,
                  e0, z0, b0, e1, z1, b1, se, sz, sb, et, zt, bt,
                  scale_v, shift_v, acc_v, sem0, sem1, semt):
        sid = lax.axis_index("s")
        wid = sid * nc + lax.axis_index("c")
        row_base = wid * rows_w
        bufs = ((e0, z0, b0), (e1, z1, b1))
        sems = (sem0, sem1)

        def start_chunk(ci, p):
            r0 = row_base + ci * chunk_rows
            return (
                pltpu.async_copy(e_hbm.at[pl.ds(r0, chunk_rows)], se.at[sid, p], sems[p]),
                pltpu.async_copy(z_hbm.at[pl.ds(r0, chunk_rows)], sz.at[sid, p], sems[p]),
                pltpu.async_copy(b_hbm.at[pl.ds(r0, chunk_rows)], sb.at[sid, p], sems[p]),
            )

        in_flight = start_chunk(0, 0)

        pltpu.sync_copy(scale_hbm, scale_v)
        pltpu.sync_copy(shift_hbm, shift_v)

        def zero_body(i, _):
            acc_v[pl.ds(i * LANES, LANES)] = jnp.zeros((LANES,), jnp.float32)
            return 0

        lax.fori_loop(0, N_STRUCTURES // LANES, zero_body, 0)

        def process(e_v, z_v, b_v, nvec):
            @plsc.parallel_loop(0, nvec, 1, unroll=8)
            def _(j):
                r = j // (ROW // LANES)
                c = (j % (ROW // LANES)) * LANES
                e = e_v[r, pl.ds(c, LANES)]
                z = z_v[r, pl.ds(c, LANES)]
                b = b_v[r, pl.ds(c, LANES)]
                sc = plsc.load_gather(scale_v, [z])
                sh = plsc.load_gather(shift_v, [z])
                plsc.addupdate_scatter(acc_v, [b], e * sc + sh)

        for ci in range(n_chunks):
            p = ci % 2
            for d in in_flight:
                d.wait()
            if ci + 1 < n_chunks:
                in_flight = start_chunk(ci + 1, (ci + 1) % 2)
            e_v, z_v, b_v = bufs[p]
            pltpu.sync_copy(se.at[sid, p], e_v)
            pltpu.sync_copy(sz.at[sid, p], z_v)
            pltpu.sync_copy(sb.at[sid, p], b_v)
            process(e_v, z_v, b_v, vec_per_chunk)

        if tail_rows:
            @pl.when(wid < tail_rows)
            def _():
                tr = nw * rows_w + wid
                d1 = pltpu.async_copy(e_hbm.at[pl.ds(tr, 1)], et, semt)
                d2 = pltpu.async_copy(z_hbm.at[pl.ds(tr, 1)], zt, semt)
                d3 = pltpu.async_copy(b_hbm.at[pl.ds(tr, 1)], bt, semt)
                d1.wait()
                d2.wait()
                d3.wait()
                process(et, zt, bt, ROW // LANES)

        pltpu.sync_copy(acc_v, out_hbm.at[wid])

    return sc_kernel, nw


def _sum_partials_body(p_ref, o_ref):
    o_ref[:] = jnp.sum(p_ref[:], axis=0)


def kernel(local_energies, Z, batch, shift, scale):
    n_atoms = local_energies.shape[0]
    scale_p = jnp.zeros((TAB_PAD,), jnp.float32).at[: scale.shape[0]].set(scale)
    shift_p = jnp.zeros((TAB_PAD,), jnp.float32).at[: shift.shape[0]].set(shift)
    e2 = local_energies.reshape(-1, ROW)
    z2 = Z.reshape(-1, ROW)
    b2 = batch.reshape(-1, ROW)

    sc_kernel, nw = _make_sc_partials(n_atoms)
    partials = sc_kernel(e2, z2, b2, scale_p, shift_p)

    total = pl.pallas_call(
        _sum_partials_body,
        out_shape=jax.ShapeDtypeStruct((N_STRUCTURES,), jnp.float32),
    )(partials)
    return total


# speedup vs baseline: 280.9987x; 1.1509x over previous
import functools

import jax
import jax.numpy as jnp
from jax import lax
from jax.experimental import pallas as pl
from jax.experimental.pallas import tpu as pltpu
from jax.experimental.pallas import tpu_sc as plsc

N_STRUCTURES = 1024
TAB_PAD = 128
LANES = 16


def _make_sc_partials(n_atoms):
    info = plsc.get_sparse_core_info()
    nc, ns = info.num_cores, info.num_subcores
    nw = nc * ns
    per_w = n_atoms // nw
    assert per_w * nw == n_atoms and per_w % LANES == 0

    chunk = per_w
    for cand in (10000, 6400, 3200, 1600, 800, 400, 80, 16):
        if per_w % cand == 0:
            chunk = cand
            break
    n_chunks = per_w // chunk
    vec_per_chunk = chunk // LANES
    unroll = 1
    for u in (8, 5, 4, 2):
        if vec_per_chunk % u == 0:
            unroll = u
            break

    mesh = plsc.VectorSubcoreMesh(core_axis_name="c", subcore_axis_name="s")

    @functools.partial(
        pl.kernel,
        mesh=mesh,
        out_type=jax.ShapeDtypeStruct((nw, N_STRUCTURES), jnp.float32),
        compiler_params=pltpu.CompilerParams(needs_layout_passes=False),
        scratch_types=[
            pltpu.VMEM((chunk,), jnp.float32),
            pltpu.VMEM((chunk,), jnp.int32),
            pltpu.VMEM((chunk,), jnp.int32),
            pltpu.VMEM((chunk,), jnp.float32),
            pltpu.VMEM((chunk,), jnp.int32),
            pltpu.VMEM((chunk,), jnp.int32),
            pltpu.VMEM((TAB_PAD,), jnp.float32),
            pltpu.VMEM((TAB_PAD,), jnp.float32),
            pltpu.VMEM((N_STRUCTURES,), jnp.float32),
            pltpu.SemaphoreType.DMA,
            pltpu.SemaphoreType.DMA,
            pltpu.SemaphoreType.DMA,
            pltpu.SemaphoreType.DMA,
            pltpu.SemaphoreType.DMA,
            pltpu.SemaphoreType.DMA,
        ],
    )
    def sc_kernel(e_hbm, z_hbm, b_hbm, scale_hbm, shift_hbm, out_hbm,
                  e0, z0, b0, e1, z1, b1, scale_v, shift_v, acc_v,
                  se0, sz0, sb0, se1, sz1, sb1):
        wid = lax.axis_index("s") * nc + lax.axis_index("c")
        base = wid * per_w
        bufs = ((e0, z0, b0), (e1, z1, b1))
        sems = ((se0, sz0, sb0), (se1, sz1, sb1))

        def start_chunk(ci, p):
            st = base + ci * chunk
            return (
                pltpu.async_copy(e_hbm.at[pl.ds(st, chunk)], bufs[p][0], sems[p][0]),
                pltpu.async_copy(z_hbm.at[pl.ds(st, chunk)], bufs[p][1], sems[p][1]),
                pltpu.async_copy(b_hbm.at[pl.ds(st, chunk)], bufs[p][2], sems[p][2]),
            )

        in_flight = start_chunk(0, 0)

        pltpu.sync_copy(scale_hbm, scale_v)
        pltpu.sync_copy(shift_hbm, shift_v)

        def zero_body(i, _):
            acc_v[pl.ds(i * LANES, LANES)] = jnp.zeros((LANES,), jnp.float32)
            return 0

        lax.fori_loop(0, N_STRUCTURES // LANES, zero_body, 0)

        for ci in range(n_chunks):
            p = ci % 2
            for d in in_flight:
                d.wait()
            if ci + 1 < n_chunks:
                in_flight = start_chunk(ci + 1, (ci + 1) % 2)
            e_v, z_v, b_v = bufs[p]

            @plsc.parallel_loop(0, chunk, LANES, unroll=unroll)
            def _(off, e_v=e_v, z_v=z_v, b_v=b_v):
                e = e_v[pl.ds(off, LANES)]
                z = z_v[pl.ds(off, LANES)]
                b = b_v[pl.ds(off, LANES)]
                sc = plsc.load_gather(scale_v, [z])
                sh = plsc.load_gather(shift_v, [z])
                plsc.addupdate_scatter(acc_v, [b], e * sc + sh)

        pltpu.sync_copy(acc_v, out_hbm.at[wid])

    return sc_kernel, nw


def _sum_partials_body(p_ref, o_ref):
    o_ref[:] = jnp.sum(p_ref[:], axis=0)


def kernel(local_energies, Z, batch, shift, scale):
    n_atoms = local_energies.shape[0]
    scale_p = jnp.zeros((TAB_PAD,), jnp.float32).at[: scale.shape[0]].set(scale)
    shift_p = jnp.zeros((TAB_PAD,), jnp.float32).at[: shift.shape[0]].set(shift)

    sc_kernel, nw = _make_sc_partials(n_atoms)
    partials = sc_kernel(local_energies, Z, batch, scale_p, shift_p)

    total = pl.pallas_call(
        _sum_partials_body,
        out_shape=jax.ShapeDtypeStruct((N_STRUCTURES,), jnp.float32),
    )(partials)
    return total
